# Initial kernel scaffold; baseline (speedup 1.0000x reference)
import jax
import jax.numpy as jnp
from jax.experimental import pallas as pl


def kernel(x, edge_index, edge_attr, state, batch, bond_batch, params):
    raise NotImplementedError("write your pallas kernel here")


# speedup vs baseline: 4.8726x; 4.8726x over previous
import functools
import jax
import jax.numpy as jnp
from jax import lax
from jax.experimental import pallas as pl
from jax.experimental.pallas import tpu as pltpu
from jax.experimental.pallas import tpu_sc as plsc

N = 10000
E = 320000
G = 16
NA = N + G
NAP = 10112
W = 128
NP2 = 10112
NW = 32
CH = E // NW
K = 80
NWIN = CH // K
RPT = NAP // 16
F32 = jnp.float32


def _bd(a, b):
    z = jnp.zeros((32, 32), F32)
    return jnp.concatenate(
        [jnp.concatenate([a, z], 1), jnp.concatenate([z, b], 1)], 0)


def _a0_body(st, wpu, bpu, wgu, wub, u0_o, gu_o, ub_o):
    u0 = jnp.maximum(st[...] @ wpu[...] + bpu[0], 0.0)
    u0_o[...] = u0
    gu_o[...] = u0 @ wgu[...]
    ub_o[...] = u0 @ wub[...]


def _a1_body(x, wpn, bpn, ws, wd, n0_o, t_o):
    n0 = jnp.maximum(x[...] @ wpn[...] + bpn[0], 0.0)
    n0_o[...] = n0
    t_o[:, 0:64] = n0 @ ws[...]
    t_o[:, 64:128] = n0 @ wd[...]


def _a2_body(ea, bb, wpe, bpe, wa, be, gu, wq, bq, p_o, q8_o):
    e0 = jnp.maximum(ea[...] @ wpe[...] + bpe[0], 0.0)
    oh = (bb[...] == lax.broadcasted_iota(jnp.int32, (512, 16), 1)).astype(F32)
    p_o[...] = e0 @ wa[...] + be[0] + oh @ gu[...]
    q8_o[...] = e0 @ wq[...] + bq[0]


def _b1_body(acc, n0, bat, wn1, wn2, bn, ub, wpn2, bpn2, ms, e6, s8_o, un_o):
    accs = acc[:, 0:W] + acc[:, W:2 * W]
    deg = jnp.maximum(accs[:, 64:65], 1.0)
    agg = accs[:, 0:64] / deg
    oh = (bat[...] == lax.broadcasted_iota(jnp.int32, (1000, 16), 1)).astype(F32)
    n0v = n0[...]
    n1 = jnp.maximum(n0v @ wn1[...] + agg @ wn2[...] + oh @ ub[...] + bn[0], 0.0)
    n_g = n1 + n0v
    n0p8 = jnp.maximum(n_g @ wpn2[...] + bpn2[0], 0.0)
    s8_o[...] = n0p8 @ ms[...] + deg @ e6[0:1]

    @pl.when(pl.program_id(0) == 0)
    def _():
        un_o[...] = jnp.zeros_like(un_o)

    contrib = lax.dot_general(oh, n1, (((0,), (0,)), ((), ())))
    cnt = jnp.sum(oh, axis=0)[:, None]
    un_o[:, 0:64] += contrib
    un_o[:, 64:65] += cnt


def _b2_body(accg, un, u0, wu1, wu2, wu3, bu, wpu2, bpu2, mu, badd, uu8_o):
    accs = accg[:, 0:W] + accg[:, W:2 * W]
    ecnt = jnp.maximum(accs[:, 64:65], 1.0)
    ue = accs[:, 0:64] / ecnt
    ncnt = jnp.maximum(un[:, 64:65], 1.0)
    unv = un[:, 0:64] / ncnt
    u0v = u0[...]
    u1 = jnp.maximum(
        u0v @ wu1[...] + ue @ wu2[...] + unv @ wu3[...] + bu[0], 0.0)
    u_g = u1 + u0v
    u0p8 = jnp.maximum(u_g @ wpu2[...] + bpu2[0], 0.0)
    uu8_o[...] = u0p8 @ mu[...] + badd[0]


def _d_body(acc2, s8, bat, uu8, dcon, h_o):
    deg = jnp.maximum(s8[:, 6:7], 1.0)
    aggp = jnp.sum(acc2[...], axis=1, keepdims=True) / deg
    n0p = s8[:, 4:5]
    oh = (bat[...] == lax.broadcasted_iota(jnp.int32, (1000, 16), 1)).astype(F32)
    ut = oh @ uu8[:, 2:3]
    d = dcon[...]
    n1p = jnp.maximum(d[0, 0] * n0p + d[0, 1] * aggp + ut + d[0, 2], 0.0)
    h_o[...] = jnp.broadcast_to(n1p + n0p, (1000, 8))


def _full(arr_shape):
    return pl.BlockSpec(arr_shape, lambda *_: tuple(0 for _ in arr_shape))


def _rows(blk, width):
    return pl.BlockSpec((blk, width), lambda i: (i, 0))


_mesh = plsc.VectorSubcoreMesh(core_axis_name="c", subcore_axis_name="s")


@functools.partial(
    pl.kernel,
    out_type=[jax.ShapeDtypeStruct((E * 8,), F32),
              jax.ShapeDtypeStruct((2, NAP, W), F32)],
    mesh=_mesh,
    compiler_params=pltpu.CompilerParams(needs_layout_passes=False),
    scratch_types=[
        pltpu.VMEM((K * 64,), F32),
        pltpu.VMEM((K * 8,), F32),
        pltpu.VMEM((K, 128), F32),
        pltpu.VMEM((K, 128), F32),
        pltpu.VMEM((K, W), F32),
        pltpu.VMEM((K * 8,), F32),
        pltpu.VMEM((K,), jnp.int32),
        pltpu.VMEM((K,), jnp.int32),
        pltpu.VMEM((K,), jnp.int32),
        pltpu.VMEM((64,), F32),
        pltpu.VMEM_SHARED((NAP, W), F32),
        pltpu.SemaphoreType.DMA,
        pltpu.SemaphoreType.DMA,
    ],
)
def _sc1(p_hbm, q8_hbm, t_hbm, src_hbm, dst_hbm, bbn_hbm, wvec_hbm,
         z80_hbm, ep_out, acc_out,
         p_v, q_v, gs_v, gd_v, sbuf, ep_v, src_v, dst_v, bbn_v, w_v, acc,
         sem1, sem2):
    cid = lax.axis_index("c")
    sid = lax.axis_index("s")
    wid = sid * 2 + cid
    pltpu.sync_copy(z80_hbm, acc.at[pl.ds(sid * RPT, RPT)])
    pltpu.sync_copy(z80_hbm.at[pl.ds(0, K)], sbuf)
    pltpu.sync_copy(wvec_hbm, w_v)
    plsc.subcore_barrier()

    iota = lax.iota(jnp.int32, 16)
    o16 = jnp.full((16,), 1, jnp.int32)
    c64 = jnp.full((16,), 64, jnp.int32)
    ones_f = jnp.ones((16,), F32)

    def win_body(w, carry):
        base = wid * CH + w * K
        cps = [pltpu.async_copy(p_hbm.at[pl.ds(base * 64, K * 64)], p_v, sem1),
               pltpu.async_copy(q8_hbm.at[pl.ds(base * 8, K * 8)], q_v, sem1),
               pltpu.async_copy(src_hbm.at[pl.ds(base, K)], src_v, sem1),
               pltpu.async_copy(dst_hbm.at[pl.ds(base, K)], dst_v, sem1),
               pltpu.async_copy(bbn_hbm.at[pl.ds(base, K)], bbn_v, sem1)]
        for c in cps:
            c.wait()
        g1 = pltpu.async_copy(t_hbm.at[src_v], gs_v, sem2)
        g2 = pltpu.async_copy(t_hbm.at[dst_v], gd_v, sem2)
        g1.wait()
        g2.wait()

        def grp(g, c2):
            row = iota + g * 16
            row64 = row * 64
            row8 = row * 8
            dii = jnp.zeros((16,), F32)
            dij = jnp.zeros((16,), F32)
            for j in range(64):
                colv = jnp.full((16,), j, jnp.int32)
                cold = jnp.full((16,), 64 + j, jnp.int32)
                pz = plsc.load_gather(p_v, [row64 + colv])
                gz = plsc.load_gather(gs_v, [row, colv])
                dz = plsc.load_gather(gd_v, [row, cold])
                e1 = jnp.maximum(pz + gz + dz, 0.0)
                plsc.store_scatter(sbuf, [row, colv], e1)
                wb = plsc.load_gather(w_v, [colv])
                if j < 32:
                    dii = dii + e1 * wb
                else:
                    dij = dij + e1 * wb
            plsc.store_scatter(sbuf, [row, c64], ones_f)
            q0 = plsc.load_gather(q_v, [row8])
            q1 = plsc.load_gather(q_v, [row8 + o16])
            plsc.store_scatter(ep_v, [row8], jnp.maximum(dii + q0, 0.0))
            plsc.store_scatter(ep_v, [row8 + o16], jnp.maximum(dij + q1, 0.0))
            return c2

        lax.fori_loop(0, K // 16, grp, 0)
        pltpu.sync_copy(sbuf, acc.at[dst_v], add=True)
        pltpu.sync_copy(sbuf, acc.at[bbn_v], add=True)
        pltpu.sync_copy(ep_v, ep_out.at[pl.ds(base * 8, K * 8)])
        return carry

    lax.fori_loop(0, NWIN, win_body, 0)
    plsc.subcore_barrier()
    pltpu.sync_copy(acc.at[pl.ds(sid * RPT, RPT)],
                    acc_out.at[cid, pl.ds(sid * RPT, RPT)])


@functools.partial(
    pl.kernel,
    out_type=[jax.ShapeDtypeStruct((E * 8,), F32),
              jax.ShapeDtypeStruct((NW * NP2,), F32)],
    mesh=_mesh,
    compiler_params=pltpu.CompilerParams(needs_layout_passes=False),
    scratch_types=[
        pltpu.VMEM((K * 8,), F32),
        pltpu.VMEM((K,), jnp.int32),
        pltpu.VMEM((K,), jnp.int32),
        pltpu.VMEM((K,), jnp.int32),
        pltpu.VMEM((N * 8,), F32),
        pltpu.VMEM((128,), F32),
        pltpu.VMEM((NP2,), F32),
        pltpu.VMEM((K * 8,), F32),
        pltpu.VMEM((16,), F32),
        pltpu.SemaphoreType.DMA,
    ],
)
def _sc2(ep8_hbm, s8f_hbm, uuf_hbm, src_hbm, dst_hbm, bb_hbm, wvec2_hbm,
         zn_hbm, h_out, acc2_out,
         ep_v, src_v, dst_v, bb_v, s8f_v, uuf_v, acc2_v, h_v, w_v, sem1):
    cid = lax.axis_index("c")
    sid = lax.axis_index("s")
    wid = sid * 2 + cid
    pltpu.sync_copy(zn_hbm, acc2_v)
    pltpu.sync_copy(zn_hbm.at[pl.ds(0, K * 8)], h_v)
    pltpu.sync_copy(s8f_hbm, s8f_v)
    pltpu.sync_copy(uuf_hbm, uuf_v)
    pltpu.sync_copy(wvec2_hbm, w_v)

    iota = lax.iota(jnp.int32, 16)
    z16 = jnp.zeros((16,), jnp.int32)
    o16 = jnp.full((16,), 1, jnp.int32)
    t16 = jnp.full((16,), 2, jnp.int32)
    th16 = jnp.full((16,), 3, jnp.int32)
    w0i = plsc.load_gather(w_v, [z16])
    w0j = plsc.load_gather(w_v, [o16])

    def win_body(w, carry):
        base = wid * CH + w * K
        cps = [pltpu.async_copy(ep8_hbm.at[pl.ds(base * 8, K * 8)], ep_v, sem1),
               pltpu.async_copy(src_hbm.at[pl.ds(base, K)], src_v, sem1),
               pltpu.async_copy(dst_hbm.at[pl.ds(base, K)], dst_v, sem1),
               pltpu.async_copy(bb_hbm.at[pl.ds(base, K)], bb_v, sem1)]
        for c in cps:
            c.wait()

        def grp(g, c2):
            row = iota + g * 16
            row8 = row * 8
            srcl = plsc.load_gather(src_v, [row])
            dstl = plsc.load_gather(dst_v, [row])
            bbl = plsc.load_gather(bb_v, [row])
            ep0 = plsc.load_gather(ep_v, [row8])
            ep1 = plsc.load_gather(ep_v, [row8 + o16])
            s8s = srcl * 8
            s8d = dstl * 8
            bb8 = bbl * 8
            zi = (ep0 * w0i + plsc.load_gather(s8f_v, [s8s + z16])
                  + plsc.load_gather(s8f_v, [s8d + t16])
                  + plsc.load_gather(uuf_v, [bb8 + z16]))
            plsc.addupdate_scatter(acc2_v, [dstl], jnp.maximum(zi, 0.0))
            zj = (ep1 * w0j + plsc.load_gather(s8f_v, [s8s + o16])
                  + plsc.load_gather(s8f_v, [s8d + th16])
                  + plsc.load_gather(uuf_v, [bb8 + o16]))
            plsc.store_scatter(h_v, [row8], jnp.maximum(zj, 0.0) + ep1)
            return c2

        lax.fori_loop(0, K // 16, grp, 0)
        pltpu.sync_copy(h_v, h_out.at[pl.ds(base * 8, K * 8)])
        return carry

    lax.fori_loop(0, NWIN, win_body, 0)
    pltpu.sync_copy(acc2_v, acc2_out.at[pl.ds(wid * NP2, NP2)])


def _pad_row(v, width=None):
    width = width or v.shape[0]
    out = jnp.zeros((8, width), F32)
    return out.at[0, : v.shape[0]].set(v)


def kernel(x, edge_index, edge_attr, state, batch, bond_batch, params):
    gii, gij = params['gii'], params['gij']
    fii, fij = params['fii'], params['fij']
    src = edge_index[0]
    dst = edge_index[1]
    bb = bond_batch.astype(jnp.int32)
    bbn = bb + N
    bb2d = bb[:, None]
    bat2d = batch.astype(jnp.int32)[:, None]

    wpn_b = jnp.concatenate([gii['W_pn'], gij['W_pn']], 1)
    bpn_b = _pad_row(jnp.concatenate([gii['b_pn'], gij['b_pn']]))
    ws_b = _bd(gii['W_e'][32:64], gij['W_e'][32:64])
    wd_b = _bd(gii['W_e'][64:96], gij['W_e'][64:96])
    wpu_b = jnp.concatenate([gii['W_pu'], gij['W_pu']], 1)
    bpu_b = _pad_row(jnp.concatenate([gii['b_pu'], gij['b_pu']]))
    wgu_b = _bd(gii['W_e'][96:128], gij['W_e'][96:128])
    wub_b = _bd(gii['W_n'][64:96], gij['W_n'][64:96])
    wpe_b = jnp.concatenate([gii['W_pe'], gij['W_pe']], 1)
    bpe_b = _pad_row(jnp.concatenate([gii['b_pe'], gij['b_pe']]))
    wa_b = _bd(gii['W_e'][0:32], gij['W_e'][0:32])
    be_b = _pad_row(jnp.concatenate([gii['b_e'], gij['b_e']]))
    wq = jnp.zeros((64, 8), F32)
    wq = wq.at[0:32, 0].set(fii['W_pe'][:, 0]).at[32:64, 1].set(fij['W_pe'][:, 0])
    bq = _pad_row(jnp.array([fii['b_pe'][0], fij['b_pe'][0], 0, 0, 0, 0, 0, 0], F32))
    wvec = jnp.concatenate([fii['W_pe'][:, 0], fij['W_pe'][:, 0]])
    wn1_b = _bd(gii['W_n'][0:32], gij['W_n'][0:32])
    wn2_b = _bd(gii['W_n'][32:64], gij['W_n'][32:64])
    bn_b = _pad_row(jnp.concatenate([gii['b_n'], gij['b_n']]))
    wpn2 = jnp.zeros((64, 8), F32)
    wpn2 = wpn2.at[0:32, 0].set(fii['W_pn'][:, 0]).at[32:64, 1].set(fij['W_pn'][:, 0])
    bpn2 = _pad_row(jnp.array([fii['b_pn'][0], fij['b_pn'][0], 0, 0, 0, 0, 0, 0], F32))
    w1_ii, w2_ii = fii['W_e'][1, 0], fii['W_e'][2, 0]
    w1_ij, w2_ij = fij['W_e'][1, 0], fij['W_e'][2, 0]
    ms = jnp.zeros((8, 8), F32)
    ms = (ms.at[0, 0].set(w1_ii).at[1, 1].set(w1_ij)
            .at[0, 2].set(w2_ii).at[1, 3].set(w2_ij)
            .at[0, 4].set(1.0).at[1, 5].set(1.0))
    e6 = _pad_row(jnp.zeros((8,), F32).at[6].set(1.0))
    wu1_b = _bd(gii['W_u'][0:32], gij['W_u'][0:32])
    wu2_b = _bd(gii['W_u'][32:64], gij['W_u'][32:64])
    wu3_b = _bd(gii['W_u'][64:96], gij['W_u'][64:96])
    bu_b = _pad_row(jnp.concatenate([gii['b_u'], gij['b_u']]))
    wpu2 = jnp.zeros((64, 8), F32)
    wpu2 = wpu2.at[0:32, 0].set(fii['W_pu'][:, 0]).at[32:64, 1].set(fij['W_pu'][:, 0])
    bpu2 = _pad_row(jnp.array([fii['b_pu'][0], fij['b_pu'][0], 0, 0, 0, 0, 0, 0], F32))
    mu = jnp.zeros((8, 8), F32)
    mu = (mu.at[0, 0].set(fii['W_e'][3, 0]).at[1, 1].set(fij['W_e'][3, 0])
            .at[0, 2].set(fii['W_n'][2, 0]))
    badd = _pad_row(jnp.array([fii['b_e'][0], fij['b_e'][0], 0, 0, 0, 0, 0, 0], F32))
    wvec2 = jnp.zeros((16,), F32).at[0].set(fii['W_e'][0, 0]).at[1].set(fij['W_e'][0, 0])
    dcon = jnp.zeros((8, 128), F32)
    dcon = (dcon.at[0, 0].set(fii['W_n'][0, 0]).at[0, 1].set(fii['W_n'][1, 0])
                .at[0, 2].set(fii['b_n'][0]))
    z80 = jnp.zeros((RPT, W), F32)
    zn = jnp.zeros((NP2,), F32)

    u0_b, gu_b, ub_b = pl.pallas_call(
        _a0_body,
        grid=(1,),
        in_specs=[_full((16, 16)), _full((16, 64)), _full((8, 64)),
                  _full((64, 64)), _full((64, 64))],
        out_specs=[_full((16, 64)), _full((16, 64)), _full((16, 64))],
        out_shape=[jax.ShapeDtypeStruct((16, 64), F32)] * 3,
    )(state, wpu_b, bpu_b, wgu_b, wub_b)

    n0_b, t_t = pl.pallas_call(
        _a1_body,
        grid=(N // 1000,),
        in_specs=[_rows(1000, 128), _full((128, 64)), _full((8, 64)),
                  _full((64, 64)), _full((64, 64))],
        out_specs=[_rows(1000, 64), _rows(1000, 128)],
        out_shape=[jax.ShapeDtypeStruct((N, 64), F32),
                   jax.ShapeDtypeStruct((N, 128), F32)],
    )(x, wpn_b, bpn_b, ws_b, wd_b)

    p_t, q8_t = pl.pallas_call(
        _a2_body,
        grid=(E // 512,),
        in_specs=[_rows(512, 16), _rows(512, 1), _full((16, 64)),
                  _full((8, 64)), _full((64, 64)), _full((8, 64)),
                  _full((16, 64)), _full((64, 8)), _full((8, 8))],
        out_specs=[_rows(512, 64), _rows(512, 8)],
        out_shape=[jax.ShapeDtypeStruct((E, 64), F32),
                   jax.ShapeDtypeStruct((E, 8), F32)],
    )(edge_attr, bb2d, wpe_b, bpe_b, wa_b, be_b, gu_b, wq, bq)

    ep8, accA = _sc1(p_t.reshape(E * 64), q8_t.reshape(E * 8), t_t,
                     src, dst, bbn, wvec, z80)
    accT = jnp.transpose(accA, (1, 0, 2)).reshape(NAP, 2 * W)

    s8, un = pl.pallas_call(
        _b1_body,
        grid=(N // 1000,),
        in_specs=[_rows(1000, 2 * W), _rows(1000, 64), _rows(1000, 1),
                  _full((64, 64)), _full((64, 64)), _full((8, 64)),
                  _full((16, 64)), _full((64, 8)), _full((8, 8)),
                  _full((8, 8)), _full((8, 8))],
        out_specs=[_rows(1000, 8),
                   pl.BlockSpec((16, 128), lambda i: (0, 0))],
        out_shape=[jax.ShapeDtypeStruct((N, 8), F32),
                   jax.ShapeDtypeStruct((16, 128), F32)],
    )(accT[:N], n0_b, bat2d, wn1_b, wn2_b, bn_b, ub_b, wpn2, bpn2, ms, e6)

    (uu8,) = pl.pallas_call(
        _b2_body,
        grid=(1,),
        in_specs=[_full((16, 2 * W)), _full((16, 128)), _full((16, 64)),
                  _full((64, 64)), _full((64, 64)), _full((64, 64)),
                  _full((8, 64)), _full((64, 8)), _full((8, 8)),
                  _full((8, 8)), _full((8, 8))],
        out_specs=[_full((16, 8))],
        out_shape=[jax.ShapeDtypeStruct((16, 8), F32)],
    )(accT[N:NA], un, u0_b, wu1_b, wu2_b, wu3_b, bu_b, wpu2, bpu2, mu, badd)

    h8f, acc2 = _sc2(ep8, s8.reshape(N * 8), uu8.reshape(128), src, dst, bb,
                     wvec2, zn)
    h2 = h8f.reshape(E, 8)[:, 0:1]
    acc2t = jnp.transpose(acc2.reshape(NW, NP2)[:, 0:N], (1, 0))

    (h8,) = pl.pallas_call(
        _d_body,
        grid=(N // 1000,),
        in_specs=[_rows(1000, NW), _rows(1000, 8), _rows(1000, 1),
                  _full((16, 8)), _full((8, 128))],
        out_specs=[_rows(1000, 8)],
        out_shape=[jax.ShapeDtypeStruct((N, 8), F32)],
    )(acc2t, s8, bat2d, uu8, dcon)

    return (h8[:, 0:1], h2, edge_index)
